# baseline (speedup 1.0000x reference)
import jax
import jax.numpy as jnp
from jax.experimental import pallas as pl
from jax.experimental.pallas import tpu as pltpu

_ALPHA_CE = 0.5
_ALPHA_KD = 0.5
_TEMP = 4.0
_TOP_K = 64
_R = 32


def _loss_kernel(s_ref, t_ref, lab_ref, out_ref, y_ref, e_ref, icut_ref,
                 acc_ref):
    b = pl.program_id(0)
    i = pl.program_id(1)

    @pl.when(jnp.logical_and(b == 0, i == 0))
    def _init():
        out_ref[...] = jnp.zeros_like(out_ref)

    s = s_ref[0]
    t = t_ref[0]
    labels = lab_ref[0, 0]
    R, V = s.shape

    ti = jax.lax.bitcast_convert_type(t, jnp.int32)
    y_ref[...] = ti ^ (jax.lax.shift_right_arithmetic(ti, 31) & jnp.int32(0x7FFFFFFF))

    imin = jnp.int32(jnp.iinfo(jnp.int32).min)

    W = 128 * 128
    P = -(-V // W)
    if V < P * W:
        tpad = jnp.concatenate(
            [t, jnp.full((R, P * W - V), -jnp.inf, t.dtype)], axis=1)
    else:
        tpad = t
    fm = tpad[:, :W]
    for k in range(1, P):
        fm = jnp.maximum(fm, tpad[:, k * W:(k + 1) * W])
    w = W
    while w > 128:
        w //= 2
        fm = jnp.maximum(fm[:, :w], fm[:, w:2 * w])
    m_t = jnp.max(fm, axis=1, keepdims=True)
    fmi = jax.lax.bitcast_convert_type(fm, jnp.int32)
    yg = fmi ^ (jax.lax.shift_right_arithmetic(fmi, 31) & jnp.int32(0x7FFFFFFF))

    def cradix_body(j, lo):
        cand = lo + jax.lax.shift_left(jnp.int32(1), 31 - j)
        cnt = jnp.sum((yg >= cand).astype(jnp.int32), axis=1, keepdims=True)
        return jnp.where(cnt >= _TOP_K, cand, lo)

    yL = jax.lax.fori_loop(0, 32, cradix_body,
                           jnp.full((R, 1), imin, jnp.int32))

    mti = jax.lax.bitcast_convert_type(m_t, jnp.int32)
    yM = mti ^ (jax.lax.shift_right_arithmetic(mti, 31) & jnp.int32(0x7FFFFFFF))
    zL = yL ^ imin
    x = zL ^ (yM ^ imin)
    xf_bits = jax.lax.bitcast_convert_type(x.astype(jnp.float32), jnp.int32)
    expo = (jax.lax.shift_right_logical(xf_bits, 23) & jnp.int32(0xFF)) - 127
    d = jnp.where(x < 0, jnp.int32(31), jnp.clip(expo, 0, 31))
    d_max = jnp.max(d)
    mask = (jnp.int32(2) << d_max) - 1
    tau0 = (zL & ~mask) ^ imin
    j0 = 31 - d_max

    def radix_cond(carry):
        j, _, c_cur = carry
        return jnp.logical_and(j < 32, jnp.any(c_cur != _TOP_K))

    def radix_body(carry):
        j, tau, c_cur = carry
        cand = tau + jax.lax.shift_left(jnp.int32(1), 31 - j)
        cnt = jnp.sum((y_ref[...] >= cand).astype(jnp.int32), axis=1, keepdims=True)
        take = cnt >= _TOP_K
        return (j + 1,
                jnp.where(take, cand, tau),
                jnp.where(take, cnt, c_cur))

    c0 = jnp.full((R, 1), V, jnp.int32)
    _, tau, c_ge = jax.lax.while_loop(radix_cond, radix_body, (j0, tau0, c0))

    idx = jax.lax.broadcasted_iota(jnp.int32, (R, V), 1)
    has_tie = jnp.any(c_ge > _TOP_K)

    @pl.when(has_tie)
    def _ties():
        c_gt = jnp.sum((y_ref[...] > tau).astype(jnp.int32), axis=1, keepdims=True)
        need = _TOP_K - c_gt

        def tie_body(j, lohi):
            lo, hi = lohi
            mid = (lo + hi) >> 1
            eqm = (y_ref[...] == tau) & (idx <= mid)
            c = jnp.sum(eqm.astype(jnp.int32), axis=1, keepdims=True)
            ok = c >= need
            return (jnp.where(ok, lo, mid + 1), jnp.where(ok, mid, hi))

        lo0 = jnp.zeros((R, 1), jnp.int32)
        hi0 = jnp.full((R, 1), V - 1, jnp.int32)
        _, hi = jax.lax.fori_loop(0, 15, tie_body, (lo0, hi0))
        icut_ref[...] = jnp.broadcast_to(hi, icut_ref.shape)

    lane = jax.lax.broadcasted_iota(jnp.int32, (R, 128), 1)

    @pl.when(jnp.logical_not(has_tie))
    def _e_plain():
        dt = t - m_t
        e = jnp.where(y_ref[...] >= tau, jnp.exp(dt / _TEMP), 0.0)
        e_ref[...] = e
        Eb = jnp.sum(e, axis=1, keepdims=True)
        SAb = jnp.sum(e * dt, axis=1, keepdims=True)
        acc_ref[...] = jnp.where(lane == 0, Eb, 0.0) + jnp.where(lane == 1, SAb, 0.0)

    @pl.when(has_tie)
    def _e_tie():
        icut = icut_ref[:, 0:1]
        sel = (y_ref[...] >= tau) & ((y_ref[...] > tau) | (idx <= icut))
        dt = t - m_t
        e = jnp.where(sel, jnp.exp(dt / _TEMP), 0.0)
        e_ref[...] = e
        Eb = jnp.sum(e, axis=1, keepdims=True)
        SAb = jnp.sum(e * dt, axis=1, keepdims=True)
        acc_ref[...] = jnp.where(lane == 0, Eb, 0.0) + jnp.where(lane == 1, SAb, 0.0)

    E = acc_ref[:, 0:1]
    SA = acc_ref[:, 1:2]

    m_s = jnp.max(s, axis=1, keepdims=True)
    Zs = jnp.sum(jnp.exp(s - m_s), axis=1, keepdims=True)
    s_lab = jnp.sum(jnp.where(idx == labels, s, 0.0), axis=1, keepdims=True)
    nll = (jnp.log(Zs) + m_s) - s_lab

    ksel = e_ref[...] > 0.0
    qarg = jnp.where(ksel, (s - m_s) / _TEMP, -100.0)
    Zq = jnp.sum(jnp.exp(qarg), axis=1, keepdims=True)
    SB0 = jnp.sum(e_ref[...] * s, axis=1, keepdims=True)

    plogp = SA / (_TEMP * E) - jnp.log(E)
    plogq = (SB0 / E - m_s) / _TEMP - jnp.log(Zq)
    row_kl = plogp - plogq

    valid = labels != jnp.int32(-100)
    ce_c = jnp.where(valid, nll, 0.0)
    kd_c = jnp.where(valid, row_kl, 0.0)
    nv_c = valid.astype(jnp.float32)

    contrib = (jnp.where(lane == 0, ce_c, 0.0)
               + jnp.where(lane == 1, kd_c, 0.0)
               + jnp.where(lane == 2, nv_c, 0.0))
    out_ref[...] += contrib


def kernel(student_logits, teacher_logits, targets):
    B, S, V = student_logits.shape
    R = _R

    lab = jnp.concatenate(
        [targets[:, 1:].astype(jnp.int32),
         jnp.full((B, 1), -100, jnp.int32)], axis=1)
    if S % R:
        pad = R - S % R
        student_logits = jnp.pad(student_logits, ((0, 0), (0, pad), (0, 0)))
        teacher_logits = jnp.pad(teacher_logits, ((0, 0), (0, pad), (0, 0)))
        lab = jnp.pad(lab, ((0, 0), (0, pad)), constant_values=-100)
        S = S + pad
    lab = lab.reshape(B, S // R, R, 1)

    out = pl.pallas_call(
        _loss_kernel,
        grid=(B, S // R),
        in_specs=[
            pl.BlockSpec((1, R, V), lambda b, i: (b, i, 0)),
            pl.BlockSpec((1, R, V), lambda b, i: (b, i, 0)),
            pl.BlockSpec((1, 1, R, 1), lambda b, i: (b, i, 0, 0)),
        ],
        out_specs=pl.BlockSpec((R, 128), lambda b, i: (0, 0)),
        out_shape=jax.ShapeDtypeStruct((R, 128), jnp.float32),
        scratch_shapes=[
            pltpu.VMEM((R, V), jnp.int32),
            pltpu.VMEM((R, V), jnp.float32),
            pltpu.VMEM((R, 128), jnp.int32),
            pltpu.VMEM((R, 128), jnp.float32),
        ],
        compiler_params=pltpu.CompilerParams(
            dimension_semantics=("arbitrary", "arbitrary")),
    )(student_logits, teacher_logits, lab)

    ce_sum = jnp.sum(out[:, 0])
    kd_sum = jnp.sum(out[:, 1])
    nv = jnp.sum(out[:, 2])
    loss_ce = ce_sum / jnp.maximum(nv, 1.0)
    loss_kd = kd_sum / nv * (_TEMP ** 2)
    return _ALPHA_CE * loss_ce + _ALPHA_KD * loss_kd


# speedup vs baseline: 1.0377x; 1.0377x over previous
import jax
import jax.numpy as jnp
from jax.experimental import pallas as pl
from jax.experimental.pallas import tpu as pltpu

_ALPHA_CE = 0.5
_ALPHA_KD = 0.5
_TEMP = 4.0
_TOP_K = 64
_R = 32


def _loss_kernel(s_ref, t_ref, lab_ref, out_ref, y_ref, e_ref, icut_ref):
    b = pl.program_id(0)
    i = pl.program_id(1)

    @pl.when(jnp.logical_and(b == 0, i == 0))
    def _init():
        out_ref[...] = jnp.zeros_like(out_ref)

    s = s_ref[0]
    t = t_ref[0]
    labels = lab_ref[0, 0]
    R, V = s.shape

    ti = jax.lax.bitcast_convert_type(t, jnp.int32)
    y_ref[...] = ti ^ (jax.lax.shift_right_arithmetic(ti, 31) & jnp.int32(0x7FFFFFFF))

    imin = jnp.int32(jnp.iinfo(jnp.int32).min)

    W = 128 * 128
    P = -(-V // W)
    if V < P * W:
        tpad = jnp.concatenate(
            [t, jnp.full((R, P * W - V), -jnp.inf, t.dtype)], axis=1)
    else:
        tpad = t
    fm = tpad[:, :W]
    for k in range(1, P):
        fm = jnp.maximum(fm, tpad[:, k * W:(k + 1) * W])
    w = W
    while w > 128:
        w //= 2
        fm = jnp.maximum(fm[:, :w], fm[:, w:2 * w])
    m_t = jnp.max(fm, axis=1, keepdims=True)
    fmi = jax.lax.bitcast_convert_type(fm, jnp.int32)
    yg = fmi ^ (jax.lax.shift_right_arithmetic(fmi, 31) & jnp.int32(0x7FFFFFFF))

    def cradix_body(j, lo):
        cand = lo + jax.lax.shift_left(jnp.int32(1), 31 - j)
        cnt = jnp.sum((yg >= cand).astype(jnp.int32), axis=1, keepdims=True)
        return jnp.where(cnt >= _TOP_K, cand, lo)

    yL = jax.lax.fori_loop(0, 32, cradix_body,
                           jnp.full((R, 1), imin, jnp.int32))

    mti = jax.lax.bitcast_convert_type(m_t, jnp.int32)
    yM = mti ^ (jax.lax.shift_right_arithmetic(mti, 31) & jnp.int32(0x7FFFFFFF))
    zL = yL ^ imin
    x = zL ^ (yM ^ imin)
    xf_bits = jax.lax.bitcast_convert_type(x.astype(jnp.float32), jnp.int32)
    expo = (jax.lax.shift_right_logical(xf_bits, 23) & jnp.int32(0xFF)) - 127
    d = jnp.where(x < 0, jnp.int32(31), jnp.clip(expo, 0, 31))
    d_max = jnp.max(d)
    mask = (jnp.int32(2) << d_max) - 1
    tau0 = (zL & ~mask) ^ imin
    j0 = 31 - d_max

    def radix_cond(carry):
        j, _, c_cur = carry
        return jnp.logical_and(j < 32, jnp.any(c_cur != _TOP_K))

    def radix_body(carry):
        j, tau, c_cur = carry
        cand = tau + jax.lax.shift_left(jnp.int32(1), 31 - j)
        cnt = jnp.sum((y_ref[...] >= cand).astype(jnp.int32), axis=1, keepdims=True)
        take = cnt >= _TOP_K
        return (j + 1,
                jnp.where(take, cand, tau),
                jnp.where(take, cnt, c_cur))

    c0 = jnp.full((R, 1), V, jnp.int32)
    _, tau, c_ge = jax.lax.while_loop(radix_cond, radix_body, (j0, tau0, c0))

    idx = jax.lax.broadcasted_iota(jnp.int32, (R, V), 1)
    has_tie = jnp.any(c_ge > _TOP_K)

    @pl.when(has_tie)
    def _ties():
        c_gt = jnp.sum((y_ref[...] > tau).astype(jnp.int32), axis=1, keepdims=True)
        need = _TOP_K - c_gt

        def tie_body(j, lohi):
            lo, hi = lohi
            mid = (lo + hi) >> 1
            eqm = (y_ref[...] == tau) & (idx <= mid)
            c = jnp.sum(eqm.astype(jnp.int32), axis=1, keepdims=True)
            ok = c >= need
            return (jnp.where(ok, lo, mid + 1), jnp.where(ok, mid, hi))

        lo0 = jnp.zeros((R, 1), jnp.int32)
        hi0 = jnp.full((R, 1), V - 1, jnp.int32)
        _, hi = jax.lax.fori_loop(0, 15, tie_body, (lo0, hi0))
        icut_ref[...] = jnp.broadcast_to(hi, icut_ref.shape)

    @pl.when(jnp.logical_not(has_tie))
    def _e_plain():
        e_ref[...] = jnp.where(y_ref[...] >= tau,
                               jnp.exp((t - m_t) / _TEMP), 0.0)

    @pl.when(has_tie)
    def _e_tie():
        icut = icut_ref[:, 0:1]
        sel = (y_ref[...] >= tau) & ((y_ref[...] > tau) | (idx <= icut))
        e_ref[...] = jnp.where(sel, jnp.exp((t - m_t) / _TEMP), 0.0)

    E = jnp.sum(e_ref[...], axis=1, keepdims=True)
    SA = jnp.sum(e_ref[...] * (t - m_t), axis=1, keepdims=True)

    m_s = jnp.max(s, axis=1, keepdims=True)
    Zs = jnp.sum(jnp.exp(s - m_s), axis=1, keepdims=True)
    s_lab = jnp.sum(jnp.where(idx == labels, s, 0.0), axis=1, keepdims=True)
    nll = (jnp.log(Zs) + m_s) - s_lab

    ksel = e_ref[...] > 0.0
    qarg = jnp.where(ksel, (s - m_s) / _TEMP, -100.0)
    Zq = jnp.sum(jnp.exp(qarg), axis=1, keepdims=True)
    SB0 = jnp.sum(e_ref[...] * s, axis=1, keepdims=True)

    plogp = SA / (_TEMP * E) - jnp.log(E)
    plogq = (SB0 / E - m_s) / _TEMP - jnp.log(Zq)
    row_kl = plogp - plogq

    valid = labels != jnp.int32(-100)
    ce_c = jnp.where(valid, nll, 0.0)
    kd_c = jnp.where(valid, row_kl, 0.0)
    nv_c = valid.astype(jnp.float32)

    lane = jax.lax.broadcasted_iota(jnp.int32, (R, 128), 1)
    contrib = (jnp.where(lane == 0, ce_c, 0.0)
               + jnp.where(lane == 1, kd_c, 0.0)
               + jnp.where(lane == 2, nv_c, 0.0))
    out_ref[...] += contrib


def kernel(student_logits, teacher_logits, targets):
    B, S, V = student_logits.shape
    R = _R

    lab = jnp.concatenate(
        [targets[:, 1:].astype(jnp.int32),
         jnp.full((B, 1), -100, jnp.int32)], axis=1)
    if S % R:
        pad = R - S % R
        student_logits = jnp.pad(student_logits, ((0, 0), (0, pad), (0, 0)))
        teacher_logits = jnp.pad(teacher_logits, ((0, 0), (0, pad), (0, 0)))
        lab = jnp.pad(lab, ((0, 0), (0, pad)), constant_values=-100)
        S = S + pad
    lab = lab.reshape(B, S // R, R, 1)

    out = pl.pallas_call(
        _loss_kernel,
        grid=(B, S // R),
        in_specs=[
            pl.BlockSpec((1, R, V), lambda b, i: (b, i, 0)),
            pl.BlockSpec((1, R, V), lambda b, i: (b, i, 0)),
            pl.BlockSpec((1, 1, R, 1), lambda b, i: (b, i, 0, 0)),
        ],
        out_specs=pl.BlockSpec((R, 128), lambda b, i: (0, 0)),
        out_shape=jax.ShapeDtypeStruct((R, 128), jnp.float32),
        scratch_shapes=[
            pltpu.VMEM((R, V), jnp.int32),
            pltpu.VMEM((R, V), jnp.float32),
            pltpu.VMEM((R, 128), jnp.int32),
        ],
        compiler_params=pltpu.CompilerParams(
            dimension_semantics=("arbitrary", "arbitrary")),
    )(student_logits, teacher_logits, lab)

    ce_sum = jnp.sum(out[:, 0])
    kd_sum = jnp.sum(out[:, 1])
    nv = jnp.sum(out[:, 2])
    loss_ce = ce_sum / jnp.maximum(nv, 1.0)
    loss_kd = kd_sum / nv * (_TEMP ** 2)
    return _ALPHA_CE * loss_ce + _ALPHA_KD * loss_kd
